# Initial kernel scaffold; baseline (speedup 1.0000x reference)
import jax
import jax.numpy as jnp
from jax.experimental import pallas as pl


def kernel(x, edge_index, curvature, W_lin, b_lin, W_curv, b_curv):
    raise NotImplementedError("write your pallas kernel here")


# speedup vs baseline: 2.5666x; 2.5666x over previous
import functools

import jax
import jax.numpy as jnp
from jax import lax
from jax.experimental import pallas as pl
from jax.experimental.pallas import tpu as pltpu
from jax.experimental.pallas import tpu_sc as plsc

N = 10000
E = 320000
D = 128

NC = 2
NS = 16
NW = NC * NS
EPW = E // NW
C = 80
NCHUNK = EPW // C
NP = 10240
ROWS_PER_TILE = NP // NS
CNT_PAD = 10240
CNT_PER_TILE = CNT_PAD // NS


def _sc_scatter(row, col, curv, x, wc, bc):
  mesh = plsc.VectorSubcoreMesh(core_axis_name="c", subcore_axis_name="s")

  @functools.partial(
      pl.kernel,
      mesh=mesh,
      out_type=[
          jax.ShapeDtypeStruct((NC, NP, D), jnp.float32),
          jax.ShapeDtypeStruct((NC, CNT_PAD), jnp.float32),
      ],
      scratch_types=[
          pltpu.VMEM((C,), jnp.int32),
          pltpu.VMEM((C,), jnp.int32),
          pltpu.VMEM((C, D), jnp.float32),
          pltpu.VMEM((C,), jnp.float32),
          pltpu.VMEM((C,), jnp.float32),
          pltpu.VMEM((C,), jnp.float32),
          pltpu.VMEM((D,), jnp.float32),
          pltpu.VMEM((D,), jnp.float32),
          pltpu.VMEM((128, D), jnp.float32),
          pltpu.VMEM((CNT_PER_TILE,), jnp.float32),
          pltpu.VMEM_SHARED((NP, D), jnp.float32),
          pltpu.VMEM_SHARED((CNT_PAD,), jnp.float32),
          pltpu.SemaphoreType.DMA,
      ],
  )
  def sc_kernel(row_hbm, col_hbm, curv_hbm, wc_hbm, bc_hbm, x_hbm,
                acc_out, cnt_out,
                ridx_v, cidx_v, rows_v, cr_v, cc_v, ones_v, wc_v, bc_v,
                zrow_v, zcnt_v, acc_s, cnt_s, sem):
    cid = lax.axis_index("c")
    sid = lax.axis_index("s")

    pltpu.sync_copy(wc_hbm, wc_v)
    pltpu.sync_copy(bc_hbm, bc_v)

    zero16 = jnp.zeros((16,), jnp.float32)
    one16 = jnp.ones((16,), jnp.float32)

    def fill_ones(i, _):
      ones_v[pl.ds(i * 16, 16)] = one16
      return 0
    lax.fori_loop(0, C // 16, fill_ones, 0)

    def zrow_fill2(i, _):
      for k in range(D // 16):
        zrow_v[i, pl.ds(k * 16, 16)] = zero16
      return 0
    lax.fori_loop(0, 128, zrow_fill2, 0)

    def zcnt_fill(i, _):
      zcnt_v[pl.ds(i * 16, 16)] = zero16
      return 0
    lax.fori_loop(0, CNT_PER_TILE // 16, zcnt_fill, 0)

    for j in range(ROWS_PER_TILE // 128):
      pltpu.sync_copy(zrow_v, acc_s.at[pl.ds(sid * ROWS_PER_TILE + j * 128, 128)])
    pltpu.sync_copy(zcnt_v, cnt_s.at[pl.ds(sid * CNT_PER_TILE, CNT_PER_TILE)])
    plsc.subcore_barrier()

    base = (cid * NS + sid) * EPW

    def chunk_body(g, _):
      eb = base + g * C
      pltpu.sync_copy(row_hbm.at[pl.ds(eb, C)], ridx_v)
      pltpu.sync_copy(col_hbm.at[pl.ds(eb, C)], cidx_v)
      h_rows = pltpu.async_copy(x_hbm.at[cidx_v], rows_v, sem)
      h_cr = pltpu.async_copy(curv_hbm.at[ridx_v], cr_v, sem)
      h_cc = pltpu.async_copy(curv_hbm.at[cidx_v], cc_v, sem)
      h_rows.wait()
      h_cr.wait()
      h_cc.wait()

      def group_body(g, _):
        off = pl.multiple_of(g * 16, 16)
        dvec = jnp.abs(cr_v[pl.ds(off, 16)] - cc_v[pl.ds(off, 16)])
        for j in range(16):
          de = lax.gather(
              dvec, jnp.full((16, 1), j, jnp.int32),
              lax.GatherDimensionNumbers(offset_dims=(),
                                         collapsed_slice_dims=(0,),
                                         start_index_map=(0,)),
              (1,), mode=lax.GatherScatterMode.PROMISE_IN_BOUNDS)
          e = off + j
          for k in range(D // 16):
            sl = pl.ds(k * 16, 16)
            z = de * wc_v[sl] + bc_v[sl]
            w = 1.0 / (1.0 + jnp.exp(-z))
            rows_v[e, sl] = rows_v[e, sl] * w
        return 0
      lax.fori_loop(0, C // 16, group_body, 0)

      pltpu.sync_copy(rows_v, acc_s.at[ridx_v], add=True)
      pltpu.sync_copy(ones_v, cnt_s.at[ridx_v], add=True)
      return 0

    lax.fori_loop(0, NCHUNK, chunk_body, 0)
    plsc.subcore_barrier()

    pltpu.sync_copy(acc_s.at[pl.ds(sid * ROWS_PER_TILE, ROWS_PER_TILE)],
                    acc_out.at[cid, pl.ds(sid * ROWS_PER_TILE, ROWS_PER_TILE)])
    pltpu.sync_copy(cnt_s.at[pl.ds(sid * CNT_PER_TILE, CNT_PER_TILE)],
                    cnt_out.at[cid, pl.ds(sid * CNT_PER_TILE, CNT_PER_TILE)])

  return sc_kernel(row, col, curv, wc, bc, x)


def _tc_finish_body(acc_ref, cnt_ref, wl_ref, bl_ref, out_ref):
  acc = acc_ref[0] + acc_ref[1]
  cnt = cnt_ref[0] + cnt_ref[1]
  inv = 1.0 / jnp.maximum(cnt, 1.0)
  mean = acc * inv[:, None]
  h = lax.dot_general(mean, wl_ref[...], (((1,), (1,)), ((), ())),
                      preferred_element_type=jnp.float32)
  h = h + bl_ref[...][None, :]
  out_ref[...] = 0.5 * h * (1.0 + lax.erf(h * (2.0 ** -0.5)))


def _tc_finish(acc, cnt, W_lin, b_lin):
  return pl.pallas_call(
      _tc_finish_body,
      out_shape=jax.ShapeDtypeStruct((NP, D), jnp.float32),
  )(acc, cnt, W_lin, b_lin)


@jax.jit
def kernel(x, edge_index, curvature, W_lin, b_lin, W_curv, b_curv):
  row = edge_index[0]
  col = edge_index[1]
  wc = W_curv[:, 0]
  acc, cnt = _sc_scatter(row, col, curvature, x, wc, b_curv)
  return _tc_finish(acc, cnt, W_lin, b_lin)[:N]
